# baseline (speedup 1.0000x reference)
import functools

import jax
import jax.numpy as jnp
from jax import lax
from jax.experimental import pallas as pl
from jax.experimental.pallas import tpu as pltpu
from jax.experimental.pallas import tpu_sc as plsc

_B, _H, _MAX, _NEW, _D = 8, 8, 4096, 16, 128
_HALF = _D // 2
_BF = jnp.bfloat16
_NC, _NS, _L = 2, 16, 16
_NW = _NC * _NS
_PPW = (_B * _H) // _NW


def _tc_body(pos_ref, kc_ref, vc_ref, kco_ref, vco_ref, mask_ref):
    i = pl.program_id(0)

    @pl.when(i == 0)
    def _():
        ids = lax.broadcasted_iota(jnp.int32, (_MAX, 1), 0)
        m = jnp.ones((_MAX, 1), jnp.float32)
        for j in range(_NEW):
            m = jnp.where(ids == pos_ref[j], 0.0, m)
        mask_ref[...] = m

    m = mask_ref[...]
    kco_ref[0] = (kc_ref[0] * m).astype(_BF)
    vco_ref[0] = (vc_ref[0] * m).astype(_BF)


def _tc_stream(cache_position, kc3, vc3):
    cache_spec = pl.BlockSpec((1, _MAX, _D), lambda i: (i, 0, 0))
    return pl.pallas_call(
        _tc_body,
        grid=(_B * _H,),
        in_specs=[
            pl.BlockSpec(memory_space=pltpu.SMEM),
            cache_spec,
            cache_spec,
        ],
        out_specs=[cache_spec, cache_spec],
        out_shape=[
            jax.ShapeDtypeStruct((_B * _H, _MAX, _D), _BF),
            jax.ShapeDtypeStruct((_B * _H, _MAX, _D), _BF),
        ],
        scratch_shapes=[pltpu.VMEM((_MAX, 1), jnp.float32)],
        compiler_params=pltpu.CompilerParams(
            dimension_semantics=("arbitrary",),
        ),
    )(cache_position, kc3, vc3)


def _sc_math_rows(gk_v, cos_v, sin_v, ks_v, out_v, accc_v, accs_v, first):
    def row(r, carry):
        for c in range(4):
            cl = pl.ds(c * _L, _L)
            cr = pl.ds((c + 4) * _L, _L)
            a = gk_v[r, cl]
            bq = gk_v[r, cr]
            out_v[r, cl] = a * cos_v[r, cl] + bq * sin_v[r, cr]
            out_v[r, cr] = bq * cos_v[r, cr] - a * sin_v[r, cl]
            kl = ks_v[r, cl]
            kr = ks_v[r, cr]
            pc_l = a * kl
            pc_r = bq * kr
            ps_l = -(a * kr)
            ps_r = bq * kl
            if first:
                accc_v[r, cl] = pc_l
                accc_v[r, cr] = pc_r
                accs_v[r, cl] = ps_l
                accs_v[r, cr] = ps_r
            else:
                plsc.addupdate(accc_v.at[r, cl], pc_l)
                plsc.addupdate(accc_v.at[r, cr], pc_r)
                plsc.addupdate(accs_v.at[r, cl], ps_l)
                plsc.addupdate(accs_v.at[r, cr], ps_r)
        return carry

    lax.fori_loop(0, _NEW, row, 0)


def _sc_add_rows(dst_v, src_v):
    def row(r, carry):
        for c in range(8):
            sl = pl.ds(c * _L, _L)
            plsc.addupdate(dst_v.at[r, sl], src_v[r, sl])
        return carry

    lax.fori_loop(0, _NEW, row, 0)


def _sc_kernel_body(kc_hbm, vc_hbm, ks_hbm, cos_hbm, sin_hbm, pos2_hbm,
                    gks_hbm, gvs_hbm, gcos_hbm, gsin_hbm,
                    idx_v, gk_v, gv_v, ks_v, cos_v, sin_v, out_v,
                    accc_v, accs_v, tmp_v, sh_cos, sh_sin, sem):
    cid = lax.axis_index("c")
    sid = lax.axis_index("s")
    w = cid * _NS + sid
    b = w // 4

    pltpu.sync_copy(cos_hbm.at[pl.ds(b * _NEW, _NEW)], cos_v)
    pltpu.sync_copy(sin_hbm.at[pl.ds(b * _NEW, _NEW)], sin_v)

    for q in range(_PPW):
        p = w * _PPW + q
        pltpu.sync_copy(pos2_hbm.at[p], idx_v)
        pltpu.async_copy(kc_hbm.at[idx_v], gk_v, sem).wait()
        pltpu.async_copy(vc_hbm.at[idx_v], gv_v, sem).wait()
        pltpu.sync_copy(gv_v, gvs_hbm.at[pl.ds(p * _NEW, _NEW)])
        pltpu.sync_copy(ks_hbm.at[pl.ds(p * _NEW, _NEW)], ks_v)
        _sc_math_rows(gk_v, cos_v, sin_v, ks_v, out_v, accc_v, accs_v,
                      first=(q == 0))
        pltpu.sync_copy(out_v, gks_hbm.at[pl.ds(p * _NEW, _NEW)])

    pltpu.sync_copy(accc_v, sh_cos.at[sid])
    pltpu.sync_copy(accs_v, sh_sin.at[sid])
    plsc.subcore_barrier()

    @pl.when(sid % 4 == 0)
    def _leader():
        for i in range(1, 4):
            pltpu.sync_copy(sh_cos.at[sid + i], tmp_v)
            _sc_add_rows(accc_v, tmp_v)
            pltpu.sync_copy(sh_sin.at[sid + i], tmp_v)
            _sc_add_rows(accs_v, tmp_v)
        pltpu.sync_copy(accc_v, gcos_hbm.at[pl.ds(b * _NEW, _NEW)])
        pltpu.sync_copy(accs_v, gsin_hbm.at[pl.ds(b * _NEW, _NEW)])


def _sc_small(kc_flat, vc_flat, ks_flat, cos_flat, sin_flat, pos2):
    run = pl.kernel(
        _sc_kernel_body,
        out_type=[
            jax.ShapeDtypeStruct((_B * _H * _NEW, _D), jnp.float32),
            jax.ShapeDtypeStruct((_B * _H * _NEW, _D), jnp.float32),
            jax.ShapeDtypeStruct((_B * _NEW, _D), jnp.float32),
            jax.ShapeDtypeStruct((_B * _NEW, _D), jnp.float32),
        ],
        mesh=plsc.VectorSubcoreMesh(
            core_axis_name="c", subcore_axis_name="s",
            num_cores=_NC, num_subcores=_NS),
        scratch_types=[
            pltpu.VMEM((_NEW,), jnp.int32),
            pltpu.VMEM((_NEW, _D), jnp.float32),
            pltpu.VMEM((_NEW, _D), jnp.float32),
            pltpu.VMEM((_NEW, _D), jnp.float32),
            pltpu.VMEM((_NEW, _D), jnp.float32),
            pltpu.VMEM((_NEW, _D), jnp.float32),
            pltpu.VMEM((_NEW, _D), jnp.float32),
            pltpu.VMEM((_NEW, _D), jnp.float32),
            pltpu.VMEM((_NEW, _D), jnp.float32),
            pltpu.VMEM((_NEW, _D), jnp.float32),
            pltpu.VMEM_SHARED((_NS, _NEW, _D), jnp.float32),
            pltpu.VMEM_SHARED((_NS, _NEW, _D), jnp.float32),
            pltpu.SemaphoreType.DMA,
        ],
    )
    return run(kc_flat, vc_flat, ks_flat, cos_flat, sin_flat, pos2)


@jax.jit
def kernel(grad_key_cache, grad_value_cache, key_states, cos, sin,
           cache_position):
    kc3 = grad_key_cache.reshape(_B * _H, _MAX, _D)
    vc3 = grad_value_cache.reshape(_B * _H, _MAX, _D)
    kco, vco = _tc_stream(cache_position, kc3, vc3)

    pos2 = (cache_position[None, :].astype(jnp.int32)
            + (jnp.arange(_B * _H, dtype=jnp.int32) * _MAX)[:, None])
    gks_f, gvs_f, gcos_f, gsin_f = _sc_small(
        grad_key_cache.reshape(_B * _H * _MAX, _D),
        grad_value_cache.reshape(_B * _H * _MAX, _D),
        key_states.reshape(_B * _H * _NEW, _D),
        cos.reshape(_B * _NEW, _D),
        sin.reshape(_B * _NEW, _D),
        pos2,
    )

    return (
        gks_f.astype(_BF).reshape(_B, _H, _NEW, _D),
        gvs_f.astype(_BF).reshape(_B, _H, _NEW, _D),
        gcos_f.astype(_BF).reshape(_B, _NEW, _D),
        gsin_f.astype(_BF).reshape(_B, _NEW, _D),
        kco.reshape(_B, _H, _MAX, _D),
        vco.reshape(_B, _H, _MAX, _D),
    )


# speedup vs baseline: 1.0044x; 1.0044x over previous
import functools

import jax
import jax.numpy as jnp
from jax import lax
from jax.experimental import pallas as pl
from jax.experimental.pallas import tpu as pltpu
from jax.experimental.pallas import tpu_sc as plsc

_B, _H, _MAX, _NEW, _D = 8, 8, 4096, 16, 128
_HALF = _D // 2
_BF = jnp.bfloat16
_NC, _NS, _L = 2, 16, 16
_NW = _NC * _NS
_PPW = (_B * _H) // _NW


def _tc_body(pos_ref, kc_ref, vc_ref, kco_ref, vco_ref, mask_ref):
    i = pl.program_id(0)

    @pl.when(i == 0)
    def _():
        ids = lax.broadcasted_iota(jnp.int32, (_MAX, 1), 0)
        m = jnp.ones((_MAX, 1), jnp.float32)
        for j in range(_NEW):
            m = jnp.where(ids == pos_ref[j], 0.0, m)
        mask_ref[...] = m

    m = mask_ref[...]
    kco_ref[0] = (kc_ref[0] * m).astype(_BF)
    vco_ref[0] = (vc_ref[0] * m).astype(_BF)


def _tc_stream(cache_position, kc3, vc3):
    cache_spec = pl.BlockSpec((1, _MAX, _D), lambda i: (i, 0, 0))
    return pl.pallas_call(
        _tc_body,
        grid=(_B * _H,),
        in_specs=[
            pl.BlockSpec(memory_space=pltpu.SMEM),
            cache_spec,
            cache_spec,
        ],
        out_specs=[cache_spec, cache_spec],
        out_shape=[
            jax.ShapeDtypeStruct((_B * _H, _MAX, _D), _BF),
            jax.ShapeDtypeStruct((_B * _H, _MAX, _D), _BF),
        ],
        scratch_shapes=[pltpu.VMEM((_MAX, 1), jnp.float32)],
        compiler_params=pltpu.CompilerParams(
            dimension_semantics=("arbitrary",),
        ),
    )(cache_position, kc3, vc3)


def _sc_math_rows(gk_v, cos_v, sin_v, ks_v, out_v, accc_v, accs_v, first,
                  row_off):
    def row(r, carry):
        for c in range(4):
            cl = pl.ds(c * _L, _L)
            cr = pl.ds((c + 4) * _L, _L)
            a = gk_v[r, cl]
            bq = gk_v[r, cr]
            out_v[row_off + r, cl] = a * cos_v[r, cl] + bq * sin_v[r, cr]
            out_v[row_off + r, cr] = bq * cos_v[r, cr] - a * sin_v[r, cl]
            kl = ks_v[row_off + r, cl]
            kr = ks_v[row_off + r, cr]
            pc_l = a * kl
            pc_r = bq * kr
            ps_l = -(a * kr)
            ps_r = bq * kl
            if first:
                accc_v[r, cl] = pc_l
                accc_v[r, cr] = pc_r
                accs_v[r, cl] = ps_l
                accs_v[r, cr] = ps_r
            else:
                plsc.addupdate(accc_v.at[r, cl], pc_l)
                plsc.addupdate(accc_v.at[r, cr], pc_r)
                plsc.addupdate(accs_v.at[r, cl], ps_l)
                plsc.addupdate(accs_v.at[r, cr], ps_r)
        return carry

    lax.fori_loop(0, _NEW, row, 0)


def _sc_add_rows(dst_v, src_v):
    def row(r, carry):
        for c in range(8):
            sl = pl.ds(c * _L, _L)
            plsc.addupdate(dst_v.at[r, sl], src_v[r, sl])
        return carry

    lax.fori_loop(0, _NEW, row, 0)


def _sc_kernel_body(kc_hbm, vc_hbm, ks_hbm, cos_hbm, sin_hbm, pos2_hbm,
                    gks_hbm, gvs_hbm, gcos_hbm, gsin_hbm,
                    idx2_v, gk0_v, gk1_v, gv0_v, gv1_v, ks2_v, cos_v, sin_v,
                    out2_v, accc_v, accs_v, tmp_v, sh_cos, sh_sin, sem):
    cid = lax.axis_index("c")
    sid = lax.axis_index("s")
    w = cid * _NS + sid
    b = w // 4
    p0 = w * _PPW

    pltpu.sync_copy(pos2_hbm.at[pl.ds(p0, _PPW)], idx2_v)
    c0 = pltpu.async_copy(kc_hbm.at[idx2_v.at[0]], gk0_v, sem)
    c1 = pltpu.async_copy(vc_hbm.at[idx2_v.at[0]], gv0_v, sem)
    c2 = pltpu.async_copy(kc_hbm.at[idx2_v.at[1]], gk1_v, sem)
    c3 = pltpu.async_copy(vc_hbm.at[idx2_v.at[1]], gv1_v, sem)
    pltpu.sync_copy(ks_hbm.at[pl.ds(p0 * _NEW, _PPW * _NEW)], ks2_v)
    pltpu.sync_copy(cos_hbm.at[pl.ds(b * _NEW, _NEW)], cos_v)
    pltpu.sync_copy(sin_hbm.at[pl.ds(b * _NEW, _NEW)], sin_v)
    c0.wait()
    c1.wait()
    c2.wait()
    c3.wait()

    pltpu.sync_copy(gv0_v, gvs_hbm.at[pl.ds(p0 * _NEW, _NEW)])
    pltpu.sync_copy(gv1_v, gvs_hbm.at[pl.ds((p0 + 1) * _NEW, _NEW)])
    for q, gk_v in ((0, gk0_v), (1, gk1_v)):
        _sc_math_rows(gk_v, cos_v, sin_v, ks2_v, out2_v, accc_v, accs_v,
                      first=(q == 0), row_off=q * _NEW)
    pltpu.sync_copy(out2_v, gks_hbm.at[pl.ds(p0 * _NEW, _PPW * _NEW)])

    pltpu.sync_copy(accc_v, sh_cos.at[sid])
    pltpu.sync_copy(accs_v, sh_sin.at[sid])
    plsc.subcore_barrier()

    @pl.when(sid % 4 == 0)
    def _leader():
        for i in range(1, 4):
            pltpu.sync_copy(sh_cos.at[sid + i], tmp_v)
            _sc_add_rows(accc_v, tmp_v)
            pltpu.sync_copy(sh_sin.at[sid + i], tmp_v)
            _sc_add_rows(accs_v, tmp_v)
        pltpu.sync_copy(accc_v, gcos_hbm.at[pl.ds(b * _NEW, _NEW)])
        pltpu.sync_copy(accs_v, gsin_hbm.at[pl.ds(b * _NEW, _NEW)])


def _sc_small(kc_flat, vc_flat, ks_flat, cos_flat, sin_flat, pos2):
    run = pl.kernel(
        _sc_kernel_body,
        out_type=[
            jax.ShapeDtypeStruct((_B * _H * _NEW, _D), jnp.float32),
            jax.ShapeDtypeStruct((_B * _H * _NEW, _D), jnp.float32),
            jax.ShapeDtypeStruct((_B * _NEW, _D), jnp.float32),
            jax.ShapeDtypeStruct((_B * _NEW, _D), jnp.float32),
        ],
        mesh=plsc.VectorSubcoreMesh(
            core_axis_name="c", subcore_axis_name="s",
            num_cores=_NC, num_subcores=_NS),
        scratch_types=[
            pltpu.VMEM((_PPW, _NEW), jnp.int32),
            pltpu.VMEM((_NEW, _D), jnp.float32),
            pltpu.VMEM((_NEW, _D), jnp.float32),
            pltpu.VMEM((_NEW, _D), jnp.float32),
            pltpu.VMEM((_NEW, _D), jnp.float32),
            pltpu.VMEM((_PPW * _NEW, _D), jnp.float32),
            pltpu.VMEM((_NEW, _D), jnp.float32),
            pltpu.VMEM((_NEW, _D), jnp.float32),
            pltpu.VMEM((_PPW * _NEW, _D), jnp.float32),
            pltpu.VMEM((_NEW, _D), jnp.float32),
            pltpu.VMEM((_NEW, _D), jnp.float32),
            pltpu.VMEM((_NEW, _D), jnp.float32),
            pltpu.VMEM_SHARED((_NS, _NEW, _D), jnp.float32),
            pltpu.VMEM_SHARED((_NS, _NEW, _D), jnp.float32),
            pltpu.SemaphoreType.DMA,
        ],
    )
    return run(kc_flat, vc_flat, ks_flat, cos_flat, sin_flat, pos2)


@jax.jit
def kernel(grad_key_cache, grad_value_cache, key_states, cos, sin,
           cache_position):
    pos2 = (cache_position[None, :].astype(jnp.int32)
            + (jnp.arange(_B * _H, dtype=jnp.int32) * _MAX)[:, None])
    gks_f, gvs_f, gcos_f, gsin_f = _sc_small(
        grad_key_cache.reshape(_B * _H * _MAX, _D),
        grad_value_cache.reshape(_B * _H * _MAX, _D),
        key_states.reshape(_B * _H * _NEW, _D),
        cos.reshape(_B * _NEW, _D),
        sin.reshape(_B * _NEW, _D),
        pos2,
    )

    kc3 = grad_key_cache.reshape(_B * _H, _MAX, _D)
    vc3 = grad_value_cache.reshape(_B * _H, _MAX, _D)
    kco, vco = _tc_stream(cache_position, kc3, vc3)

    return (
        gks_f.astype(_BF).reshape(_B, _H, _NEW, _D),
        gvs_f.astype(_BF).reshape(_B, _H, _NEW, _D),
        gcos_f.astype(_BF).reshape(_B, _NEW, _D),
        gsin_f.astype(_BF).reshape(_B, _NEW, _D),
        kco.reshape(_B, _H, _MAX, _D),
        vco.reshape(_B, _H, _MAX, _D),
    )
